# baseline (speedup 1.0000x reference)
import functools

import jax
import jax.numpy as jnp
from jax import lax
from jax.experimental import pallas as pl
from jax.experimental.pallas import tpu as pltpu
from jax.experimental.pallas import tpu_sc as plsc

B = 2
S = 2048
HIDDEN = 1024
FFN = 3584
E = 8
TOPK = 2

T = B * S
A = T * TOPK
BLK = 512
NB = A // BLK + E
NPAD = NB * BLK
NF = 2
FBLK = FFN // NF

NC = 2
NS = 16
NW = NC * NS
LANES = 16

CH = 512
NCH = A // CH

_SC_MESH = dict(core_axis_name="c", subcore_axis_name="s",
                num_cores=NC, num_subcores=NS)


def _meta_body(x_ref, gw_ref, logits_ref, pos1_ref, pos2_ref, wa_ref, wb_ref,
               be_ref, o_scr, r_scr):
    x = x_ref[...]
    gw = gw_ref[...]
    logits = lax.dot_general(x, gw, (((1,), (1,)), ((), ())),
                             preferred_element_type=jnp.float32)
    logits_ref[...] = logits

    p = jax.nn.softmax(logits, axis=1)
    ii = lax.broadcasted_iota(jnp.int32, (T, E), 1).astype(jnp.float32)
    m1 = jnp.max(p, axis=1, keepdims=True)
    i1 = jnp.min(jnp.where(p == m1, ii, float(E)), axis=1, keepdims=True)
    oh1 = (ii == i1).astype(jnp.float32)
    pm = jnp.where(oh1 > 0.0, -1.0, p)
    m2 = jnp.max(pm, axis=1, keepdims=True)
    i2 = jnp.min(jnp.where(pm == m2, ii, float(E)), axis=1, keepdims=True)
    oh2 = (ii == i2).astype(jnp.float32)
    ssum = m1 + m2
    wa = m1 / ssum
    wb = m2 / ssum
    wa_ref[...] = wa
    wb_ref[...] = wb

    o_scr[0:T, :] = oh1
    o_scr[T:A, :] = oh2

    tri = (lax.broadcasted_iota(jnp.int32, (CH, CH), 0)
           >= lax.broadcasted_iota(jnp.int32, (CH, CH), 1)
           ).astype(jnp.float32)

    def step(c, carry):
        blk = o_scr[pl.ds(c * CH, CH), :]
        cum = lax.dot_general(tri, blk, (((1,), (0,)), ((), ())),
                              preferred_element_type=jnp.float32) + carry
        r_scr[pl.ds(c * CH, CH), :] = jnp.sum(
            blk * (cum - 1.0), axis=1, keepdims=True)
        return carry + jnp.sum(blk, axis=0, keepdims=True)

    counts = lax.fori_loop(0, NCH, step, jnp.zeros((1, E), jnp.float32))

    bc = jnp.floor((counts + float(BLK - 1)) / float(BLK)) * float(BLK)
    lt = (lax.broadcasted_iota(jnp.int32, (E, E), 0)
          < lax.broadcasted_iota(jnp.int32, (E, E), 1)).astype(jnp.float32)
    off = lax.dot_general(bc, lt, (((1,), (0,)), ((), ())),
                          preferred_element_type=jnp.float32)

    r = r_scr[...]
    pos1 = r[0:T, :] + jnp.sum(oh1 * off, axis=1, keepdims=True)
    pos2 = r[T:A, :] + jnp.sum(oh2 * off, axis=1, keepdims=True)
    pos1_ref[...] = pos1.astype(jnp.int32)
    pos2_ref[...] = pos2.astype(jnp.int32)

    bidx = lax.broadcasted_iota(jnp.int32, (1, 64), 1).astype(jnp.float32)
    be = jnp.zeros((1, 64), jnp.float32)
    off_blocks = off / float(BLK)
    for e in range(E):
        be = be + (bidx >= off_blocks[0:1, e:e + 1]).astype(jnp.float32)
    nbu = jnp.sum(jnp.floor((counts + float(BLK - 1)) / float(BLK)),
                  axis=1, keepdims=True)
    be = jnp.where(bidx == 63.0, nbu, be - 1.0)
    be_ref[...] = be.astype(jnp.int32)


def _run_meta(x, gate_w):
    return pl.pallas_call(
        _meta_body,
        out_shape=(
            jax.ShapeDtypeStruct((T, E), jnp.float32),
            jax.ShapeDtypeStruct((T, 1), jnp.int32),
            jax.ShapeDtypeStruct((T, 1), jnp.int32),
            jax.ShapeDtypeStruct((T, 1), jnp.float32),
            jax.ShapeDtypeStruct((T, 1), jnp.float32),
            jax.ShapeDtypeStruct((1, 64), jnp.int32),
        ),
        scratch_shapes=[
            pltpu.VMEM((A, E), jnp.float32),
            pltpu.VMEM((A, 1), jnp.float32),
        ],
    )(x, gate_w)


def _scatter_body(pos1_hbm, pos2_hbm, wa_hbm, wb_hbm, tid_hbm, wts_hbm,
                  p1_v, p2_v, wa_v, wb_v, tid_v, wts_v):
    cid = lax.axis_index("c")
    sid = lax.axis_index("s")
    wid = sid * NC + cid

    @pl.when(wid == 0)
    def _():
        pltpu.sync_copy(pos1_hbm, p1_v)
        pltpu.sync_copy(pos2_hbm, p2_v)
        pltpu.sync_copy(wa_hbm, wa_v)
        pltpu.sync_copy(wb_hbm, wb_v)

        zi = jnp.zeros((LANES,), jnp.int32)
        zf = jnp.zeros((LANES,), jnp.float32)

        def zero(i, _):
            tid_v[pl.ds(i * LANES, LANES)] = zi
            wts_v[pl.ds(i * LANES, LANES)] = zf
            return 0

        lax.fori_loop(0, NPAD // LANES, zero, 0)

        base_iota = lax.broadcasted_iota(jnp.int32, (LANES,), 0)

        def scat(i, _):
            tok = base_iota + i * LANES
            idx1 = p1_v[pl.ds(i * LANES, LANES)]
            plsc.store_scatter(tid_v, [idx1], tok)
            plsc.store_scatter(wts_v, [idx1], wa_v[pl.ds(i * LANES, LANES)])
            idx2 = p2_v[pl.ds(i * LANES, LANES)]
            plsc.store_scatter(tid_v, [idx2], tok)
            plsc.store_scatter(wts_v, [idx2], wb_v[pl.ds(i * LANES, LANES)])
            return 0

        lax.fori_loop(0, T // LANES, scat, 0)

        pltpu.sync_copy(tid_v, tid_hbm)
        pltpu.sync_copy(wts_v, wts_hbm)


def _run_scatter(pos1, pos2, wa, wb):
    mesh = plsc.VectorSubcoreMesh(**_SC_MESH)
    return pl.kernel(
        _scatter_body,
        out_type=(
            jax.ShapeDtypeStruct((NPAD,), jnp.int32),
            jax.ShapeDtypeStruct((NPAD,), jnp.float32),
        ),
        mesh=mesh,
        compiler_params=pltpu.CompilerParams(needs_layout_passes=False),
        scratch_types=[
            pltpu.VMEM((T,), jnp.int32),
            pltpu.VMEM((T,), jnp.int32),
            pltpu.VMEM((T,), jnp.float32),
            pltpu.VMEM((T,), jnp.float32),
            pltpu.VMEM((NPAD,), jnp.int32),
            pltpu.VMEM((NPAD,), jnp.float32),
        ],
    )(pos1, pos2, wa, wb)


_G_PER_W = NPAD // NW
_G_CH = 96
_G_NCH = _G_PER_W // _G_CH


def _gather_body(x_hbm, tid_hbm, xs_hbm, idx_v, rows_v, sem):
    cid = lax.axis_index("c")
    sid = lax.axis_index("s")
    wid = sid * NC + cid
    base = wid * _G_PER_W
    pltpu.sync_copy(tid_hbm.at[pl.ds(base, _G_PER_W)], idx_v)
    for c in range(_G_NCH):
        pltpu.async_copy(x_hbm.at[idx_v.at[pl.ds(c * _G_CH, _G_CH)]],
                         rows_v, sem).wait()
        pltpu.sync_copy(rows_v, xs_hbm.at[pl.ds(base + c * _G_CH, _G_CH)])


def _run_gather(x, tid_sorted):
    mesh = plsc.VectorSubcoreMesh(**_SC_MESH)
    return pl.kernel(
        _gather_body,
        out_type=jax.ShapeDtypeStruct((NPAD, HIDDEN), jnp.float32),
        mesh=mesh,
        compiler_params=pltpu.CompilerParams(needs_layout_passes=False),
        scratch_types=[
            pltpu.VMEM((_G_PER_W,), jnp.int32),
            pltpu.VMEM((_G_CH, HIDDEN), jnp.float32),
            pltpu.SemaphoreType.DMA,
        ],
    )(x, tid_sorted)


def _gmm_body(be_ref, x_ref, w1_ref, w3_ref, w2_ref, wt_ref, o_ref,
              xs_scr):
    b = pl.program_id(0)
    f = pl.program_id(1)
    nbu = be_ref[63]

    @pl.when(b < nbu)
    def _():
        @pl.when(f == 0)
        def _():
            xs_scr[...] = x_ref[...].astype(jnp.bfloat16)

        xs = xs_scr[...]
        a = lax.dot_general(xs, w1_ref[0], (((1,), (1,)), ((), ())),
                            preferred_element_type=jnp.float32)
        g = a * jax.nn.sigmoid(a)
        c = (g * lax.dot_general(xs, w3_ref[0], (((1,), (1,)), ((), ())),
                                 preferred_element_type=jnp.float32)
             ).astype(jnp.bfloat16)
        part = lax.dot_general(c, w2_ref[0], (((1,), (1,)), ((), ())),
                               preferred_element_type=jnp.float32)

        @pl.when(f == 0)
        def _():
            o_ref[...] = part

        @pl.when(f > 0)
        def _():
            o_ref[...] = o_ref[...] + part

        @pl.when(f == NF - 1)
        def _():
            o_ref[...] = o_ref[...] * wt_ref[0]


def _run_gmm(be_vec, x_sorted, w1, w3, w2, wts3d):
    grid_spec = pltpu.PrefetchScalarGridSpec(
        num_scalar_prefetch=1,
        grid=(NB, NF),
        in_specs=[
            pl.BlockSpec((BLK, HIDDEN), lambda b, f, be: (b, 0)),
            pl.BlockSpec((1, FBLK, HIDDEN), lambda b, f, be: (be[b], f, 0)),
            pl.BlockSpec((1, FBLK, HIDDEN), lambda b, f, be: (be[b], f, 0)),
            pl.BlockSpec((1, HIDDEN, FBLK), lambda b, f, be: (be[b], 0, f)),
            pl.BlockSpec((1, BLK, 1), lambda b, f, be: (b, 0, 0)),
        ],
        out_specs=pl.BlockSpec((BLK, HIDDEN), lambda b, f, be: (b, 0)),
        scratch_shapes=[pltpu.VMEM((BLK, HIDDEN), jnp.bfloat16)],
    )
    return pl.pallas_call(
        _gmm_body,
        grid_spec=grid_spec,
        out_shape=jax.ShapeDtypeStruct((NPAD, HIDDEN), jnp.float32),
        compiler_params=pltpu.CompilerParams(
            dimension_semantics=("parallel", "arbitrary"),
            vmem_limit_bytes=110 * 1024 * 1024),
    )(be_vec, x_sorted, w1, w3, w2, wts3d)


_C_PER_W = T // NW
_C_CH = 16
_C_NCH = _C_PER_W // _C_CH


def _combine_body(os_hbm, p1_hbm, p2_hbm, out_hbm, p1_v, p2_v,
                  a0_v, a1_v, b0_v, b1_v, sa0, sa1, sb0, sb1):
    cid = lax.axis_index("c")
    sid = lax.axis_index("s")
    wid = sid * NC + cid
    base = wid * _C_PER_W
    pltpu.sync_copy(p1_hbm.at[pl.ds(base, _C_PER_W)], p1_v)
    pltpu.sync_copy(p2_hbm.at[pl.ds(base, _C_PER_W)], p2_v)
    abufs = (a0_v, a1_v)
    bbufs = (b0_v, b1_v)
    asems = (sa0, sa1)
    bsems = (sb0, sb1)

    def issue(c):
        da = pltpu.async_copy(os_hbm.at[p1_v.at[pl.ds(c * _C_CH, _C_CH)]],
                              abufs[c % 2], asems[c % 2])
        db = pltpu.async_copy(os_hbm.at[p2_v.at[pl.ds(c * _C_CH, _C_CH)]],
                              bbufs[c % 2], bsems[c % 2])
        return (da, db)

    descs = {0: issue(0)}
    for c in range(_C_NCH):
        if c + 1 < _C_NCH:
            descs[c + 1] = issue(c + 1)
        da, db = descs[c]
        da.wait()
        db.wait()
        av = abufs[c % 2]
        bv = bbufs[c % 2]

        def add_row(i, _):
            def add_col(j, _2):
                av[i, pl.ds(j * LANES, LANES)] = (
                    av[i, pl.ds(j * LANES, LANES)]
                    + bv[i, pl.ds(j * LANES, LANES)])
                return 0
            lax.fori_loop(0, HIDDEN // LANES, add_col, 0)
            return 0

        lax.fori_loop(0, _C_CH, add_row, 0)
        pltpu.sync_copy(av, out_hbm.at[pl.ds(base + c * _C_CH, _C_CH)])


def _run_combine(out_sorted, p1, p2):
    mesh = plsc.VectorSubcoreMesh(**_SC_MESH)
    return pl.kernel(
        _combine_body,
        out_type=jax.ShapeDtypeStruct((T, HIDDEN), jnp.float32),
        mesh=mesh,
        compiler_params=pltpu.CompilerParams(needs_layout_passes=False),
        scratch_types=[
            pltpu.VMEM((_C_PER_W,), jnp.int32),
            pltpu.VMEM((_C_PER_W,), jnp.int32),
            pltpu.VMEM((_C_CH, HIDDEN), jnp.float32),
            pltpu.VMEM((_C_CH, HIDDEN), jnp.float32),
            pltpu.VMEM((_C_CH, HIDDEN), jnp.float32),
            pltpu.VMEM((_C_CH, HIDDEN), jnp.float32),
            pltpu.SemaphoreType.DMA,
            pltpu.SemaphoreType.DMA,
            pltpu.SemaphoreType.DMA,
            pltpu.SemaphoreType.DMA,
        ],
    )(out_sorted, p1, p2)


def kernel(hidden_states, gate_w, w1, w2, w3, scales):
    x = hidden_states.reshape(T, HIDDEN)
    logits, pos1, pos2, wa, wb, be = _run_meta(x, gate_w)
    tid_sorted, wts_sorted = _run_scatter(
        pos1.reshape(T), pos2.reshape(T), wa.reshape(T), wb.reshape(T))
    x_sorted = _run_gather(x, tid_sorted)
    inv_s = (1.0 / scales)[:, None, :]
    out_sorted = _run_gmm(be.reshape(64), x_sorted,
                          (w1 * inv_s).astype(jnp.bfloat16),
                          (w3 * inv_s).astype(jnp.bfloat16),
                          w2.astype(jnp.bfloat16),
                          wts_sorted.reshape(NB, BLK, 1))
    final = _run_combine(out_sorted, pos1.reshape(T), pos2.reshape(T))
    return (final.reshape(B, S, HIDDEN), logits)


# speedup vs baseline: 1.0659x; 1.0659x over previous
import functools

import jax
import jax.numpy as jnp
from jax import lax
from jax.experimental import pallas as pl
from jax.experimental.pallas import tpu as pltpu
from jax.experimental.pallas import tpu_sc as plsc

B = 2
S = 2048
HIDDEN = 1024
FFN = 3584
E = 8
TOPK = 2

T = B * S
A = T * TOPK
BLK = 512
NB = A // BLK + E
NPAD = NB * BLK
NF = 1
FBLK = FFN // NF

NC = 2
NS = 16
NW = NC * NS
LANES = 16

CH = 512
NCH = A // CH

_SC_MESH = dict(core_axis_name="c", subcore_axis_name="s",
                num_cores=NC, num_subcores=NS)


def _meta_body(x_ref, gw_ref, logits_ref, pos1_ref, pos2_ref, wa_ref, wb_ref,
               be_ref, o_scr, r_scr):
    x = x_ref[...]
    gw = gw_ref[...]
    logits = lax.dot_general(x, gw, (((1,), (1,)), ((), ())),
                             preferred_element_type=jnp.float32)
    logits_ref[...] = logits

    p = jax.nn.softmax(logits, axis=1)
    ii = lax.broadcasted_iota(jnp.int32, (T, E), 1).astype(jnp.float32)
    m1 = jnp.max(p, axis=1, keepdims=True)
    i1 = jnp.min(jnp.where(p == m1, ii, float(E)), axis=1, keepdims=True)
    oh1 = (ii == i1).astype(jnp.float32)
    pm = jnp.where(oh1 > 0.0, -1.0, p)
    m2 = jnp.max(pm, axis=1, keepdims=True)
    i2 = jnp.min(jnp.where(pm == m2, ii, float(E)), axis=1, keepdims=True)
    oh2 = (ii == i2).astype(jnp.float32)
    ssum = m1 + m2
    wa = m1 / ssum
    wb = m2 / ssum
    wa_ref[...] = wa
    wb_ref[...] = wb

    o_scr[0:T, :] = oh1
    o_scr[T:A, :] = oh2

    tri = (lax.broadcasted_iota(jnp.int32, (CH, CH), 0)
           >= lax.broadcasted_iota(jnp.int32, (CH, CH), 1)
           ).astype(jnp.float32)

    def step(c, carry):
        blk = o_scr[pl.ds(c * CH, CH), :]
        cum = lax.dot_general(tri, blk, (((1,), (0,)), ((), ())),
                              preferred_element_type=jnp.float32) + carry
        r_scr[pl.ds(c * CH, CH), :] = jnp.sum(
            blk * (cum - 1.0), axis=1, keepdims=True)
        return carry + jnp.sum(blk, axis=0, keepdims=True)

    counts = lax.fori_loop(0, NCH, step, jnp.zeros((1, E), jnp.float32))

    bc = jnp.floor((counts + float(BLK - 1)) / float(BLK)) * float(BLK)
    lt = (lax.broadcasted_iota(jnp.int32, (E, E), 0)
          < lax.broadcasted_iota(jnp.int32, (E, E), 1)).astype(jnp.float32)
    off = lax.dot_general(bc, lt, (((1,), (0,)), ((), ())),
                          preferred_element_type=jnp.float32)

    r = r_scr[...]
    pos1 = r[0:T, :] + jnp.sum(oh1 * off, axis=1, keepdims=True)
    pos2 = r[T:A, :] + jnp.sum(oh2 * off, axis=1, keepdims=True)
    pos1_ref[...] = pos1.astype(jnp.int32)
    pos2_ref[...] = pos2.astype(jnp.int32)

    bidx = lax.broadcasted_iota(jnp.int32, (1, 64), 1).astype(jnp.float32)
    be = jnp.zeros((1, 64), jnp.float32)
    off_blocks = off / float(BLK)
    for e in range(E):
        be = be + (bidx >= off_blocks[0:1, e:e + 1]).astype(jnp.float32)
    nbu = jnp.sum(jnp.floor((counts + float(BLK - 1)) / float(BLK)),
                  axis=1, keepdims=True)
    be = jnp.where(bidx == 63.0, nbu, be - 1.0)
    be_ref[...] = be.astype(jnp.int32)


def _run_meta(x, gate_w):
    return pl.pallas_call(
        _meta_body,
        out_shape=(
            jax.ShapeDtypeStruct((T, E), jnp.float32),
            jax.ShapeDtypeStruct((T, 1), jnp.int32),
            jax.ShapeDtypeStruct((T, 1), jnp.int32),
            jax.ShapeDtypeStruct((T, 1), jnp.float32),
            jax.ShapeDtypeStruct((T, 1), jnp.float32),
            jax.ShapeDtypeStruct((1, 64), jnp.int32),
        ),
        scratch_shapes=[
            pltpu.VMEM((A, E), jnp.float32),
            pltpu.VMEM((A, 1), jnp.float32),
        ],
    )(x, gate_w)


def _scatter_body(pos1_hbm, pos2_hbm, wa_hbm, wb_hbm, tid_hbm, wts_hbm,
                  p1_v, p2_v, wa_v, wb_v, tid_v, wts_v):
    cid = lax.axis_index("c")
    sid = lax.axis_index("s")
    wid = sid * NC + cid

    @pl.when(wid == 0)
    def _():
        pltpu.sync_copy(pos1_hbm, p1_v)
        pltpu.sync_copy(pos2_hbm, p2_v)
        pltpu.sync_copy(wa_hbm, wa_v)
        pltpu.sync_copy(wb_hbm, wb_v)

        zi = jnp.zeros((LANES,), jnp.int32)
        zf = jnp.zeros((LANES,), jnp.float32)

        def zero(i, _):
            tid_v[pl.ds(i * LANES, LANES)] = zi
            wts_v[pl.ds(i * LANES, LANES)] = zf
            return 0

        lax.fori_loop(0, NPAD // LANES, zero, 0)

        base_iota = lax.broadcasted_iota(jnp.int32, (LANES,), 0)

        def scat(i, _):
            tok = base_iota + i * LANES
            idx1 = p1_v[pl.ds(i * LANES, LANES)]
            plsc.store_scatter(tid_v, [idx1], tok)
            plsc.store_scatter(wts_v, [idx1], wa_v[pl.ds(i * LANES, LANES)])
            idx2 = p2_v[pl.ds(i * LANES, LANES)]
            plsc.store_scatter(tid_v, [idx2], tok)
            plsc.store_scatter(wts_v, [idx2], wb_v[pl.ds(i * LANES, LANES)])
            return 0

        lax.fori_loop(0, T // LANES, scat, 0)

        pltpu.sync_copy(tid_v, tid_hbm)
        pltpu.sync_copy(wts_v, wts_hbm)


def _run_scatter(pos1, pos2, wa, wb):
    mesh = plsc.VectorSubcoreMesh(**_SC_MESH)
    return pl.kernel(
        _scatter_body,
        out_type=(
            jax.ShapeDtypeStruct((NPAD,), jnp.int32),
            jax.ShapeDtypeStruct((NPAD,), jnp.float32),
        ),
        mesh=mesh,
        compiler_params=pltpu.CompilerParams(needs_layout_passes=False),
        scratch_types=[
            pltpu.VMEM((T,), jnp.int32),
            pltpu.VMEM((T,), jnp.int32),
            pltpu.VMEM((T,), jnp.float32),
            pltpu.VMEM((T,), jnp.float32),
            pltpu.VMEM((NPAD,), jnp.int32),
            pltpu.VMEM((NPAD,), jnp.float32),
        ],
    )(pos1, pos2, wa, wb)


_G_PER_W = NPAD // NW
_G_CH = 64
_G_NCH = _G_PER_W // _G_CH


def _gather_body(x_hbm, tid_hbm, xs_hbm, idx_v, rows_v, sem):
    cid = lax.axis_index("c")
    sid = lax.axis_index("s")
    wid = sid * NC + cid
    base = wid * _G_PER_W
    pltpu.sync_copy(tid_hbm.at[pl.ds(base, _G_PER_W)], idx_v)
    for c in range(_G_NCH):
        pltpu.async_copy(x_hbm.at[idx_v.at[pl.ds(c * _G_CH, _G_CH)]],
                         rows_v, sem).wait()
        pltpu.sync_copy(rows_v, xs_hbm.at[pl.ds(base + c * _G_CH, _G_CH)])


def _run_gather(x, tid_sorted):
    mesh = plsc.VectorSubcoreMesh(**_SC_MESH)
    return pl.kernel(
        _gather_body,
        out_type=jax.ShapeDtypeStruct((NPAD, HIDDEN), jnp.float32),
        mesh=mesh,
        compiler_params=pltpu.CompilerParams(needs_layout_passes=False),
        scratch_types=[
            pltpu.VMEM((_G_PER_W,), jnp.int32),
            pltpu.VMEM((_G_CH, HIDDEN), jnp.float32),
            pltpu.SemaphoreType.DMA,
        ],
    )(x, tid_sorted)


def _gmm_body(be_ref, x_ref, w1_ref, w3_ref, w2_ref, wt_ref, o_ref,
              xs_scr):
    b = pl.program_id(0)
    f = pl.program_id(1)
    nbu = be_ref[63]

    @pl.when(b < nbu)
    def _():
        @pl.when(f == 0)
        def _():
            xs_scr[...] = x_ref[...].astype(jnp.bfloat16)

        xs = xs_scr[...]
        a = lax.dot_general(xs, w1_ref[0], (((1,), (1,)), ((), ())),
                            preferred_element_type=jnp.float32)
        g = a * jax.nn.sigmoid(a)
        c = (g * lax.dot_general(xs, w3_ref[0], (((1,), (1,)), ((), ())),
                                 preferred_element_type=jnp.float32)
             ).astype(jnp.bfloat16)
        part = lax.dot_general(c, w2_ref[0], (((1,), (1,)), ((), ())),
                               preferred_element_type=jnp.float32)

        @pl.when(f == 0)
        def _():
            o_ref[...] = part

        @pl.when(f > 0)
        def _():
            o_ref[...] = o_ref[...] + part

        @pl.when(f == NF - 1)
        def _():
            o_ref[...] = o_ref[...] * wt_ref[0]


def _run_gmm(be_vec, x_sorted, w1, w3, w2, wts3d):
    grid_spec = pltpu.PrefetchScalarGridSpec(
        num_scalar_prefetch=1,
        grid=(NB, NF),
        in_specs=[
            pl.BlockSpec((BLK, HIDDEN), lambda b, f, be: (b, 0)),
            pl.BlockSpec((1, FBLK, HIDDEN), lambda b, f, be: (be[b], f, 0)),
            pl.BlockSpec((1, FBLK, HIDDEN), lambda b, f, be: (be[b], f, 0)),
            pl.BlockSpec((1, HIDDEN, FBLK), lambda b, f, be: (be[b], 0, f)),
            pl.BlockSpec((1, BLK, 1), lambda b, f, be: (b, 0, 0)),
        ],
        out_specs=pl.BlockSpec((BLK, HIDDEN), lambda b, f, be: (b, 0)),
        scratch_shapes=[pltpu.VMEM((BLK, HIDDEN), jnp.bfloat16)],
    )
    return pl.pallas_call(
        _gmm_body,
        grid_spec=grid_spec,
        out_shape=jax.ShapeDtypeStruct((NPAD, HIDDEN), jnp.float32),
        compiler_params=pltpu.CompilerParams(
            dimension_semantics=("parallel", "arbitrary"),
            vmem_limit_bytes=110 * 1024 * 1024),
    )(be_vec, x_sorted, w1, w3, w2, wts3d)


_C_PER_W = T // NW
_C_CH = 16
_C_NCH = _C_PER_W // _C_CH


def _combine_body(os_hbm, p1_hbm, p2_hbm, out_hbm, p1_v, p2_v,
                  a0_v, a1_v, b0_v, b1_v, sa0, sa1, sb0, sb1):
    cid = lax.axis_index("c")
    sid = lax.axis_index("s")
    wid = sid * NC + cid
    base = wid * _C_PER_W
    pltpu.sync_copy(p1_hbm.at[pl.ds(base, _C_PER_W)], p1_v)
    pltpu.sync_copy(p2_hbm.at[pl.ds(base, _C_PER_W)], p2_v)
    abufs = (a0_v, a1_v)
    bbufs = (b0_v, b1_v)
    asems = (sa0, sa1)
    bsems = (sb0, sb1)

    def issue(c):
        da = pltpu.async_copy(os_hbm.at[p1_v.at[pl.ds(c * _C_CH, _C_CH)]],
                              abufs[c % 2], asems[c % 2])
        db = pltpu.async_copy(os_hbm.at[p2_v.at[pl.ds(c * _C_CH, _C_CH)]],
                              bbufs[c % 2], bsems[c % 2])
        return (da, db)

    descs = {0: issue(0)}
    for c in range(_C_NCH):
        if c + 1 < _C_NCH:
            descs[c + 1] = issue(c + 1)
        da, db = descs[c]
        da.wait()
        db.wait()
        av = abufs[c % 2]
        bv = bbufs[c % 2]

        def add_row(i, _):
            def add_col(j, _2):
                av[i, pl.ds(j * LANES, LANES)] = (
                    av[i, pl.ds(j * LANES, LANES)]
                    + bv[i, pl.ds(j * LANES, LANES)])
                return 0
            lax.fori_loop(0, HIDDEN // LANES, add_col, 0)
            return 0

        lax.fori_loop(0, _C_CH, add_row, 0)
        pltpu.sync_copy(av, out_hbm.at[pl.ds(base + c * _C_CH, _C_CH)])


def _run_combine(out_sorted, p1, p2):
    mesh = plsc.VectorSubcoreMesh(**_SC_MESH)
    return pl.kernel(
        _combine_body,
        out_type=jax.ShapeDtypeStruct((T, HIDDEN), jnp.float32),
        mesh=mesh,
        compiler_params=pltpu.CompilerParams(needs_layout_passes=False),
        scratch_types=[
            pltpu.VMEM((_C_PER_W,), jnp.int32),
            pltpu.VMEM((_C_PER_W,), jnp.int32),
            pltpu.VMEM((_C_CH, HIDDEN), jnp.float32),
            pltpu.VMEM((_C_CH, HIDDEN), jnp.float32),
            pltpu.VMEM((_C_CH, HIDDEN), jnp.float32),
            pltpu.VMEM((_C_CH, HIDDEN), jnp.float32),
            pltpu.SemaphoreType.DMA,
            pltpu.SemaphoreType.DMA,
            pltpu.SemaphoreType.DMA,
            pltpu.SemaphoreType.DMA,
        ],
    )(out_sorted, p1, p2)


def kernel(hidden_states, gate_w, w1, w2, w3, scales):
    x = hidden_states.reshape(T, HIDDEN)
    logits, pos1, pos2, wa, wb, be = _run_meta(x, gate_w)
    tid_sorted, wts_sorted = _run_scatter(
        pos1.reshape(T), pos2.reshape(T), wa.reshape(T), wb.reshape(T))
    x_sorted = _run_gather(x, tid_sorted)
    inv_s = (1.0 / scales)[:, None, :]
    out_sorted = _run_gmm(be.reshape(64), x_sorted,
                          (w1 * inv_s).astype(jnp.bfloat16),
                          (w3 * inv_s).astype(jnp.bfloat16),
                          w2.astype(jnp.bfloat16),
                          wts_sorted.reshape(NB, BLK, 1))
    final = _run_combine(out_sorted, pos1.reshape(T), pos2.reshape(T))
    return (final.reshape(B, S, HIDDEN), logits)


# speedup vs baseline: 1.1071x; 1.0386x over previous
import functools

import jax
import jax.numpy as jnp
from jax import lax
from jax.experimental import pallas as pl
from jax.experimental.pallas import tpu as pltpu
from jax.experimental.pallas import tpu_sc as plsc

B = 2
S = 2048
HIDDEN = 1024
FFN = 3584
E = 8
TOPK = 2

T = B * S
A = T * TOPK
BLK = 512
NB = A // BLK + E
NPAD = NB * BLK
NF = 1
FBLK = FFN // NF

NC = 2
NS = 16
NW = NC * NS
LANES = 16

CH = 512
NCH = A // CH

_SC_MESH = dict(core_axis_name="c", subcore_axis_name="s",
                num_cores=NC, num_subcores=NS)


def _meta_body(x_ref, gw_ref, logits_ref, pos1_ref, pos2_ref, wa_ref, wb_ref,
               be_ref, xi_ref, o_scr, r_scr):
    x = x_ref[...]
    bits = lax.bitcast_convert_type(x, jnp.int32)
    rb = lax.shift_right_logical(
        bits + 0x7FFF + jnp.bitwise_and(lax.shift_right_logical(bits, 16), 1),
        16)
    lo = rb[:, 0:HIDDEN // 2]
    hi = rb[:, HIDDEN // 2:HIDDEN]
    xi_ref[...] = jnp.bitwise_or(lo, lax.shift_left(hi, 16))
    gw = gw_ref[...]
    logits = lax.dot_general(x, gw, (((1,), (1,)), ((), ())),
                             preferred_element_type=jnp.float32)
    logits_ref[...] = logits

    p = jax.nn.softmax(logits, axis=1)
    ii = lax.broadcasted_iota(jnp.int32, (T, E), 1).astype(jnp.float32)
    m1 = jnp.max(p, axis=1, keepdims=True)
    i1 = jnp.min(jnp.where(p == m1, ii, float(E)), axis=1, keepdims=True)
    oh1 = (ii == i1).astype(jnp.float32)
    pm = jnp.where(oh1 > 0.0, -1.0, p)
    m2 = jnp.max(pm, axis=1, keepdims=True)
    i2 = jnp.min(jnp.where(pm == m2, ii, float(E)), axis=1, keepdims=True)
    oh2 = (ii == i2).astype(jnp.float32)
    ssum = m1 + m2
    wa = m1 / ssum
    wb = m2 / ssum
    wa_ref[...] = wa
    wb_ref[...] = wb

    o_scr[0:T, :] = oh1
    o_scr[T:A, :] = oh2

    tri = (lax.broadcasted_iota(jnp.int32, (CH, CH), 0)
           >= lax.broadcasted_iota(jnp.int32, (CH, CH), 1)
           ).astype(jnp.float32)

    def step(c, carry):
        blk = o_scr[pl.ds(c * CH, CH), :]
        cum = lax.dot_general(tri, blk, (((1,), (0,)), ((), ())),
                              preferred_element_type=jnp.float32) + carry
        r_scr[pl.ds(c * CH, CH), :] = jnp.sum(
            blk * (cum - 1.0), axis=1, keepdims=True)
        return carry + jnp.sum(blk, axis=0, keepdims=True)

    counts = lax.fori_loop(0, NCH, step, jnp.zeros((1, E), jnp.float32))

    bc = jnp.floor((counts + float(BLK - 1)) / float(BLK)) * float(BLK)
    lt = (lax.broadcasted_iota(jnp.int32, (E, E), 0)
          < lax.broadcasted_iota(jnp.int32, (E, E), 1)).astype(jnp.float32)
    off = lax.dot_general(bc, lt, (((1,), (0,)), ((), ())),
                          preferred_element_type=jnp.float32)

    r = r_scr[...]
    pos1 = r[0:T, :] + jnp.sum(oh1 * off, axis=1, keepdims=True)
    pos2 = r[T:A, :] + jnp.sum(oh2 * off, axis=1, keepdims=True)
    pos1_ref[...] = pos1.astype(jnp.int32)
    pos2_ref[...] = pos2.astype(jnp.int32)

    bidx = lax.broadcasted_iota(jnp.int32, (1, 64), 1).astype(jnp.float32)
    be = jnp.zeros((1, 64), jnp.float32)
    off_blocks = off / float(BLK)
    for e in range(E):
        be = be + (bidx >= off_blocks[0:1, e:e + 1]).astype(jnp.float32)
    nbu = jnp.sum(jnp.floor((counts + float(BLK - 1)) / float(BLK)),
                  axis=1, keepdims=True)
    be = jnp.where(bidx == 63.0, nbu, be - 1.0)
    be_ref[...] = be.astype(jnp.int32)


def _run_meta(x, gate_w):
    return pl.pallas_call(
        _meta_body,
        out_shape=(
            jax.ShapeDtypeStruct((T, E), jnp.float32),
            jax.ShapeDtypeStruct((T, 1), jnp.int32),
            jax.ShapeDtypeStruct((T, 1), jnp.int32),
            jax.ShapeDtypeStruct((T, 1), jnp.float32),
            jax.ShapeDtypeStruct((T, 1), jnp.float32),
            jax.ShapeDtypeStruct((1, 64), jnp.int32),
            jax.ShapeDtypeStruct((T, HIDDEN // 2), jnp.int32),
        ),
        scratch_shapes=[
            pltpu.VMEM((A, E), jnp.float32),
            pltpu.VMEM((A, 1), jnp.float32),
        ],
    )(x, gate_w)


def _scatter_body(pos1_hbm, pos2_hbm, wa_hbm, wb_hbm, tid_hbm, wts_hbm,
                  p1_v, p2_v, wa_v, wb_v, tid_v, wts_v):
    cid = lax.axis_index("c")
    sid = lax.axis_index("s")
    wid = sid * NC + cid

    @pl.when(wid == 0)
    def _():
        pltpu.sync_copy(pos1_hbm, p1_v)
        pltpu.sync_copy(pos2_hbm, p2_v)
        pltpu.sync_copy(wa_hbm, wa_v)
        pltpu.sync_copy(wb_hbm, wb_v)

        zi = jnp.zeros((LANES,), jnp.int32)
        zf = jnp.zeros((LANES,), jnp.float32)

        def zero(i, _):
            tid_v[pl.ds(i * LANES, LANES)] = zi
            wts_v[pl.ds(i * LANES, LANES)] = zf
            return 0

        lax.fori_loop(0, NPAD // LANES, zero, 0)

        base_iota = lax.broadcasted_iota(jnp.int32, (LANES,), 0)

        def scat(i, _):
            tok = base_iota + i * LANES
            idx1 = p1_v[pl.ds(i * LANES, LANES)]
            plsc.store_scatter(tid_v, [idx1], tok)
            plsc.store_scatter(wts_v, [idx1], wa_v[pl.ds(i * LANES, LANES)])
            idx2 = p2_v[pl.ds(i * LANES, LANES)]
            plsc.store_scatter(tid_v, [idx2], tok)
            plsc.store_scatter(wts_v, [idx2], wb_v[pl.ds(i * LANES, LANES)])
            return 0

        lax.fori_loop(0, T // LANES, scat, 0)

        pltpu.sync_copy(tid_v, tid_hbm)
        pltpu.sync_copy(wts_v, wts_hbm)


def _run_scatter(pos1, pos2, wa, wb):
    mesh = plsc.VectorSubcoreMesh(**_SC_MESH)
    return pl.kernel(
        _scatter_body,
        out_type=(
            jax.ShapeDtypeStruct((NPAD,), jnp.int32),
            jax.ShapeDtypeStruct((NPAD,), jnp.float32),
        ),
        mesh=mesh,
        compiler_params=pltpu.CompilerParams(needs_layout_passes=False),
        scratch_types=[
            pltpu.VMEM((T,), jnp.int32),
            pltpu.VMEM((T,), jnp.int32),
            pltpu.VMEM((T,), jnp.float32),
            pltpu.VMEM((T,), jnp.float32),
            pltpu.VMEM((NPAD,), jnp.int32),
            pltpu.VMEM((NPAD,), jnp.float32),
        ],
    )(pos1, pos2, wa, wb)


_G_PER_W = NPAD // NW
_G_CH = 64
_G_NCH = _G_PER_W // _G_CH


def _gather_body(x_hbm, tid_hbm, xs_hbm, idx_v, rows_v, sem):
    cid = lax.axis_index("c")
    sid = lax.axis_index("s")
    wid = sid * NC + cid
    base = wid * _G_PER_W
    pltpu.sync_copy(tid_hbm.at[pl.ds(base, _G_PER_W)], idx_v)
    for c in range(_G_NCH):
        pltpu.async_copy(x_hbm.at[idx_v.at[pl.ds(c * _G_CH, _G_CH)]],
                         rows_v, sem).wait()
        pltpu.sync_copy(rows_v, xs_hbm.at[pl.ds(base + c * _G_CH, _G_CH)])


def _run_gather(xi, tid_sorted):
    mesh = plsc.VectorSubcoreMesh(**_SC_MESH)
    return pl.kernel(
        _gather_body,
        out_type=jax.ShapeDtypeStruct((NPAD, HIDDEN // 2), jnp.int32),
        mesh=mesh,
        compiler_params=pltpu.CompilerParams(needs_layout_passes=False),
        scratch_types=[
            pltpu.VMEM((_G_PER_W,), jnp.int32),
            pltpu.VMEM((_G_CH, HIDDEN // 2), jnp.int32),
            pltpu.SemaphoreType.DMA,
        ],
    )(xi, tid_sorted)


def _gmm_body(be_ref, x_ref, w1_ref, w3_ref, w2_ref, wt_ref, o_ref,
              xs_scr):
    b = pl.program_id(0)
    f = pl.program_id(1)
    nbu = be_ref[63]

    @pl.when(b < nbu)
    def _():
        @pl.when(f == 0)
        def _():
            xi = x_ref[...]
            lo = lax.bitcast_convert_type(lax.shift_left(xi, 16),
                                          jnp.float32)
            hi = lax.bitcast_convert_type(
                jnp.bitwise_and(xi, jnp.int32(-65536)), jnp.float32)
            xs_scr[...] = jnp.concatenate([lo, hi],
                                          axis=1).astype(jnp.bfloat16)

        xs = xs_scr[...]
        a = lax.dot_general(xs, w1_ref[0], (((1,), (1,)), ((), ())),
                            preferred_element_type=jnp.float32)
        g = a * jax.nn.sigmoid(a)
        c = (g * lax.dot_general(xs, w3_ref[0], (((1,), (1,)), ((), ())),
                                 preferred_element_type=jnp.float32)
             ).astype(jnp.bfloat16)
        part = lax.dot_general(c, w2_ref[0], (((1,), (1,)), ((), ())),
                               preferred_element_type=jnp.float32)

        @pl.when(f == 0)
        def _():
            o_ref[...] = part

        @pl.when(f > 0)
        def _():
            o_ref[...] = o_ref[...] + part

        @pl.when(f == NF - 1)
        def _():
            o_ref[...] = o_ref[...] * wt_ref[0]


def _run_gmm(be_vec, x_sorted, w1, w3, w2, wts3d):
    grid_spec = pltpu.PrefetchScalarGridSpec(
        num_scalar_prefetch=1,
        grid=(NB, NF),
        in_specs=[
            pl.BlockSpec((BLK, HIDDEN // 2), lambda b, f, be: (b, 0)),
            pl.BlockSpec((1, FBLK, HIDDEN), lambda b, f, be: (be[b], f, 0)),
            pl.BlockSpec((1, FBLK, HIDDEN), lambda b, f, be: (be[b], f, 0)),
            pl.BlockSpec((1, HIDDEN, FBLK), lambda b, f, be: (be[b], 0, f)),
            pl.BlockSpec((1, BLK, 1), lambda b, f, be: (b, 0, 0)),
        ],
        out_specs=pl.BlockSpec((BLK, HIDDEN), lambda b, f, be: (b, 0)),
        scratch_shapes=[pltpu.VMEM((BLK, HIDDEN), jnp.bfloat16)],
    )
    return pl.pallas_call(
        _gmm_body,
        grid_spec=grid_spec,
        out_shape=jax.ShapeDtypeStruct((NPAD, HIDDEN), jnp.float32),
        compiler_params=pltpu.CompilerParams(
            dimension_semantics=("parallel", "arbitrary"),
            vmem_limit_bytes=110 * 1024 * 1024),
    )(be_vec, x_sorted, w1, w3, w2, wts3d)


_C_PER_W = T // NW
_C_CH = 16
_C_NCH = _C_PER_W // _C_CH


def _combine_body(os_hbm, p1_hbm, p2_hbm, out_hbm, p1_v, p2_v,
                  a0_v, a1_v, b0_v, b1_v, sa0, sa1, sb0, sb1):
    cid = lax.axis_index("c")
    sid = lax.axis_index("s")
    wid = sid * NC + cid
    base = wid * _C_PER_W
    pltpu.sync_copy(p1_hbm.at[pl.ds(base, _C_PER_W)], p1_v)
    pltpu.sync_copy(p2_hbm.at[pl.ds(base, _C_PER_W)], p2_v)
    abufs = (a0_v, a1_v)
    bbufs = (b0_v, b1_v)
    asems = (sa0, sa1)
    bsems = (sb0, sb1)

    def issue(c):
        da = pltpu.async_copy(os_hbm.at[p1_v.at[pl.ds(c * _C_CH, _C_CH)]],
                              abufs[c % 2], asems[c % 2])
        db = pltpu.async_copy(os_hbm.at[p2_v.at[pl.ds(c * _C_CH, _C_CH)]],
                              bbufs[c % 2], bsems[c % 2])
        return (da, db)

    descs = {0: issue(0)}
    for c in range(_C_NCH):
        if c + 1 < _C_NCH:
            descs[c + 1] = issue(c + 1)
        da, db = descs[c]
        da.wait()
        db.wait()
        av = abufs[c % 2]
        bv = bbufs[c % 2]

        def add_row(i, _):
            def add_col(j, _2):
                av[i, pl.ds(j * LANES, LANES)] = (
                    av[i, pl.ds(j * LANES, LANES)]
                    + bv[i, pl.ds(j * LANES, LANES)])
                return 0
            lax.fori_loop(0, HIDDEN // LANES, add_col, 0)
            return 0

        lax.fori_loop(0, _C_CH, add_row, 0)
        pltpu.sync_copy(av, out_hbm.at[pl.ds(base + c * _C_CH, _C_CH)])


def _run_combine(out_sorted, p1, p2):
    mesh = plsc.VectorSubcoreMesh(**_SC_MESH)
    return pl.kernel(
        _combine_body,
        out_type=jax.ShapeDtypeStruct((T, HIDDEN), jnp.float32),
        mesh=mesh,
        compiler_params=pltpu.CompilerParams(needs_layout_passes=False),
        scratch_types=[
            pltpu.VMEM((_C_PER_W,), jnp.int32),
            pltpu.VMEM((_C_PER_W,), jnp.int32),
            pltpu.VMEM((_C_CH, HIDDEN), jnp.float32),
            pltpu.VMEM((_C_CH, HIDDEN), jnp.float32),
            pltpu.VMEM((_C_CH, HIDDEN), jnp.float32),
            pltpu.VMEM((_C_CH, HIDDEN), jnp.float32),
            pltpu.SemaphoreType.DMA,
            pltpu.SemaphoreType.DMA,
            pltpu.SemaphoreType.DMA,
            pltpu.SemaphoreType.DMA,
        ],
    )(out_sorted, p1, p2)


def kernel(hidden_states, gate_w, w1, w2, w3, scales):
    x = hidden_states.reshape(T, HIDDEN)
    logits, pos1, pos2, wa, wb, be, xi = _run_meta(x, gate_w)
    tid_sorted, wts_sorted = _run_scatter(
        pos1.reshape(T), pos2.reshape(T), wa.reshape(T), wb.reshape(T))
    x_sorted = _run_gather(xi, tid_sorted)
    inv_s = (1.0 / scales)[:, None, :]
    out_sorted = _run_gmm(be.reshape(64), x_sorted,
                          (w1 * inv_s).astype(jnp.bfloat16),
                          (w3 * inv_s).astype(jnp.bfloat16),
                          w2.astype(jnp.bfloat16),
                          wts_sorted.reshape(NB, BLK, 1))
    final = _run_combine(out_sorted, pos1.reshape(T), pos2.reshape(T))
    return (final.reshape(B, S, HIDDEN), logits)


# speedup vs baseline: 1.1247x; 1.0159x over previous
import functools

import jax
import jax.numpy as jnp
from jax import lax
from jax.experimental import pallas as pl
from jax.experimental.pallas import tpu as pltpu
from jax.experimental.pallas import tpu_sc as plsc

B = 2
S = 2048
HIDDEN = 1024
FFN = 3584
E = 8
TOPK = 2

T = B * S
A = T * TOPK
BLK = 512
NB = A // BLK + E
NPAD = NB * BLK
NF = 1
FBLK = FFN // NF

NC = 2
NS = 16
NW = NC * NS
LANES = 16

CH = 512
NCH = A // CH

_SC_MESH = dict(core_axis_name="c", subcore_axis_name="s",
                num_cores=NC, num_subcores=NS)


def _meta_body(x_ref, gw_ref, logits_ref, pos1_ref, pos2_ref, wa_ref, wb_ref,
               be_ref, xi_ref, o_scr, r_scr):
    x = x_ref[...]
    bits = lax.bitcast_convert_type(x, jnp.int32)
    rb = lax.shift_right_logical(
        bits + 0x7FFF + jnp.bitwise_and(lax.shift_right_logical(bits, 16), 1),
        16)
    lo = rb[:, 0:HIDDEN // 2]
    hi = rb[:, HIDDEN // 2:HIDDEN]
    xi_ref[...] = jnp.bitwise_or(lo, lax.shift_left(hi, 16))
    gw = gw_ref[...]
    logits = lax.dot_general(x, gw, (((1,), (1,)), ((), ())),
                             preferred_element_type=jnp.float32)
    logits_ref[...] = logits

    p = jax.nn.softmax(logits, axis=1)
    ii = lax.broadcasted_iota(jnp.int32, (T, E), 1).astype(jnp.float32)
    m1 = jnp.max(p, axis=1, keepdims=True)
    i1 = jnp.min(jnp.where(p == m1, ii, float(E)), axis=1, keepdims=True)
    oh1 = (ii == i1).astype(jnp.float32)
    pm = jnp.where(oh1 > 0.0, -1.0, p)
    m2 = jnp.max(pm, axis=1, keepdims=True)
    i2 = jnp.min(jnp.where(pm == m2, ii, float(E)), axis=1, keepdims=True)
    oh2 = (ii == i2).astype(jnp.float32)
    ssum = m1 + m2
    wa = m1 / ssum
    wb = m2 / ssum
    wa_ref[...] = wa
    wb_ref[...] = wb

    o_scr[0:T, :] = oh1
    o_scr[T:A, :] = oh2

    tri = (lax.broadcasted_iota(jnp.int32, (CH, CH), 0)
           >= lax.broadcasted_iota(jnp.int32, (CH, CH), 1)
           ).astype(jnp.float32)

    def step(c, carry):
        blk = o_scr[pl.ds(c * CH, CH), :]
        cum = lax.dot_general(tri, blk, (((1,), (0,)), ((), ())),
                              preferred_element_type=jnp.float32) + carry
        r_scr[pl.ds(c * CH, CH), :] = jnp.sum(
            blk * (cum - 1.0), axis=1, keepdims=True)
        return carry + jnp.sum(blk, axis=0, keepdims=True)

    counts = lax.fori_loop(0, NCH, step, jnp.zeros((1, E), jnp.float32))

    bc = jnp.floor((counts + float(BLK - 1)) / float(BLK)) * float(BLK)
    lt = (lax.broadcasted_iota(jnp.int32, (E, E), 0)
          < lax.broadcasted_iota(jnp.int32, (E, E), 1)).astype(jnp.float32)
    off = lax.dot_general(bc, lt, (((1,), (0,)), ((), ())),
                          preferred_element_type=jnp.float32)

    r = r_scr[...]
    pos1 = r[0:T, :] + jnp.sum(oh1 * off, axis=1, keepdims=True)
    pos2 = r[T:A, :] + jnp.sum(oh2 * off, axis=1, keepdims=True)
    pos1_ref[...] = pos1.astype(jnp.int32)
    pos2_ref[...] = pos2.astype(jnp.int32)

    bidx = lax.broadcasted_iota(jnp.int32, (1, 64), 1).astype(jnp.float32)
    be = jnp.zeros((1, 64), jnp.float32)
    off_blocks = off / float(BLK)
    for e in range(E):
        be = be + (bidx >= off_blocks[0:1, e:e + 1]).astype(jnp.float32)
    nbu = jnp.sum(jnp.floor((counts + float(BLK - 1)) / float(BLK)),
                  axis=1, keepdims=True)
    be = jnp.where(bidx == 63.0, nbu, be - 1.0)
    be_ref[...] = be.astype(jnp.int32)


def _run_meta(x, gate_w):
    return pl.pallas_call(
        _meta_body,
        out_shape=(
            jax.ShapeDtypeStruct((T, E), jnp.float32),
            jax.ShapeDtypeStruct((T, 1), jnp.int32),
            jax.ShapeDtypeStruct((T, 1), jnp.int32),
            jax.ShapeDtypeStruct((T, 1), jnp.float32),
            jax.ShapeDtypeStruct((T, 1), jnp.float32),
            jax.ShapeDtypeStruct((1, 64), jnp.int32),
            jax.ShapeDtypeStruct((T, HIDDEN // 2), jnp.int32),
        ),
        scratch_shapes=[
            pltpu.VMEM((A, E), jnp.float32),
            pltpu.VMEM((A, 1), jnp.float32),
        ],
    )(x, gate_w)


_G_PER_W = NPAD // NW
_G_CH = 64
_G_NCH = _G_PER_W // _G_CH
HID2 = HIDDEN // 2


def _dispatch_body(pos1_hbm, pos2_hbm, wa_hbm, wb_hbm, xi_hbm,
                   xs_hbm, wts_hbm,
                   p1_v, p2_v, wa_v, wb_v, tid_v, wts_v, tid_sh,
                   idx_v, rows_v, sem):
    cid = lax.axis_index("c")
    sid = lax.axis_index("s")
    wid = sid * NC + cid

    @pl.when(sid == 0)
    def _():
        pltpu.sync_copy(pos1_hbm, p1_v)
        pltpu.sync_copy(pos2_hbm, p2_v)
        pltpu.sync_copy(wa_hbm, wa_v)
        pltpu.sync_copy(wb_hbm, wb_v)

        zi = jnp.zeros((LANES,), jnp.int32)
        zf = jnp.zeros((LANES,), jnp.float32)

        def zero(i, _):
            tid_v[pl.ds(i * LANES, LANES)] = zi
            wts_v[pl.ds(i * LANES, LANES)] = zf
            return 0

        lax.fori_loop(0, NPAD // LANES, zero, 0)

        base_iota = lax.broadcasted_iota(jnp.int32, (LANES,), 0)

        def scat(i, _):
            tok = base_iota + i * LANES
            idx1 = p1_v[pl.ds(i * LANES, LANES)]
            plsc.store_scatter(tid_v, [idx1], tok)
            plsc.store_scatter(wts_v, [idx1], wa_v[pl.ds(i * LANES, LANES)])
            idx2 = p2_v[pl.ds(i * LANES, LANES)]
            plsc.store_scatter(tid_v, [idx2], tok)
            plsc.store_scatter(wts_v, [idx2], wb_v[pl.ds(i * LANES, LANES)])
            return 0

        lax.fori_loop(0, T // LANES, scat, 0)

        pltpu.sync_copy(tid_v, tid_sh)

        @pl.when(cid == 0)
        def _():
            pltpu.sync_copy(wts_v, wts_hbm)

    plsc.subcore_barrier()

    base = wid * _G_PER_W
    pltpu.sync_copy(tid_sh.at[pl.ds(base, _G_PER_W)], idx_v)
    for c in range(_G_NCH):
        pltpu.async_copy(xi_hbm.at[idx_v.at[pl.ds(c * _G_CH, _G_CH)]],
                         rows_v, sem).wait()
        pltpu.sync_copy(rows_v, xs_hbm.at[pl.ds(base + c * _G_CH, _G_CH)])


def _run_dispatch(pos1, pos2, wa, wb, xi):
    mesh = plsc.VectorSubcoreMesh(**_SC_MESH)
    return pl.kernel(
        _dispatch_body,
        out_type=(
            jax.ShapeDtypeStruct((NPAD, HID2), jnp.int32),
            jax.ShapeDtypeStruct((NPAD,), jnp.float32),
        ),
        mesh=mesh,
        compiler_params=pltpu.CompilerParams(needs_layout_passes=False),
        scratch_types=[
            pltpu.VMEM((T,), jnp.int32),
            pltpu.VMEM((T,), jnp.int32),
            pltpu.VMEM((T,), jnp.float32),
            pltpu.VMEM((T,), jnp.float32),
            pltpu.VMEM((NPAD,), jnp.int32),
            pltpu.VMEM((NPAD,), jnp.float32),
            pltpu.VMEM_SHARED((NPAD,), jnp.int32),
            pltpu.VMEM((_G_PER_W,), jnp.int32),
            pltpu.VMEM((_G_CH, HID2), jnp.int32),
            pltpu.SemaphoreType.DMA,
        ],
    )(pos1, pos2, wa, wb, xi)


def _gmm_body(be_ref, x_ref, w1_ref, w3_ref, w2_ref, wt_ref, o_ref,
              xs_scr):
    b = pl.program_id(0)
    f = pl.program_id(1)
    nbu = be_ref[63]

    @pl.when(b < nbu)
    def _():
        @pl.when(f == 0)
        def _():
            xi = x_ref[...]
            lo = lax.bitcast_convert_type(lax.shift_left(xi, 16),
                                          jnp.float32)
            hi = lax.bitcast_convert_type(
                jnp.bitwise_and(xi, jnp.int32(-65536)), jnp.float32)
            xs_scr[...] = jnp.concatenate([lo, hi],
                                          axis=1).astype(jnp.bfloat16)

        xs = xs_scr[...]
        a = lax.dot_general(xs, w1_ref[0], (((1,), (1,)), ((), ())),
                            preferred_element_type=jnp.float32)
        g = a * jax.nn.sigmoid(a)
        c = (g * lax.dot_general(xs, w3_ref[0], (((1,), (1,)), ((), ())),
                                 preferred_element_type=jnp.float32)
             ).astype(jnp.bfloat16)
        part = lax.dot_general(c, w2_ref[0], (((1,), (1,)), ((), ())),
                               preferred_element_type=jnp.float32)

        @pl.when(f == 0)
        def _():
            o_ref[...] = part

        @pl.when(f > 0)
        def _():
            o_ref[...] = o_ref[...] + part

        @pl.when(f == NF - 1)
        def _():
            o_ref[...] = o_ref[...] * wt_ref[0]


def _run_gmm(be_vec, x_sorted, w1, w3, w2, wts3d):
    grid_spec = pltpu.PrefetchScalarGridSpec(
        num_scalar_prefetch=1,
        grid=(NB, NF),
        in_specs=[
            pl.BlockSpec((BLK, HIDDEN // 2), lambda b, f, be: (b, 0)),
            pl.BlockSpec((1, FBLK, HIDDEN), lambda b, f, be: (be[b], f, 0)),
            pl.BlockSpec((1, FBLK, HIDDEN), lambda b, f, be: (be[b], f, 0)),
            pl.BlockSpec((1, HIDDEN, FBLK), lambda b, f, be: (be[b], 0, f)),
            pl.BlockSpec((1, BLK, 1), lambda b, f, be: (b, 0, 0)),
        ],
        out_specs=pl.BlockSpec((BLK, HIDDEN), lambda b, f, be: (b, 0)),
        scratch_shapes=[pltpu.VMEM((BLK, HIDDEN), jnp.bfloat16)],
    )
    return pl.pallas_call(
        _gmm_body,
        grid_spec=grid_spec,
        out_shape=jax.ShapeDtypeStruct((NPAD, HIDDEN), jnp.float32),
        compiler_params=pltpu.CompilerParams(
            dimension_semantics=("parallel", "arbitrary"),
            vmem_limit_bytes=110 * 1024 * 1024),
    )(be_vec, x_sorted, w1, w3, w2, wts3d)


_C_PER_W = T // NW
_C_CH = 16
_C_NCH = _C_PER_W // _C_CH


def _combine_body(os_hbm, p1_hbm, p2_hbm, out_hbm, p1_v, p2_v,
                  a0_v, a1_v, b0_v, b1_v, sa0, sa1, sb0, sb1):
    cid = lax.axis_index("c")
    sid = lax.axis_index("s")
    wid = sid * NC + cid
    base = wid * _C_PER_W
    pltpu.sync_copy(p1_hbm.at[pl.ds(base, _C_PER_W)], p1_v)
    pltpu.sync_copy(p2_hbm.at[pl.ds(base, _C_PER_W)], p2_v)
    abufs = (a0_v, a1_v)
    bbufs = (b0_v, b1_v)
    asems = (sa0, sa1)
    bsems = (sb0, sb1)

    def issue(c):
        da = pltpu.async_copy(os_hbm.at[p1_v.at[pl.ds(c * _C_CH, _C_CH)]],
                              abufs[c % 2], asems[c % 2])
        db = pltpu.async_copy(os_hbm.at[p2_v.at[pl.ds(c * _C_CH, _C_CH)]],
                              bbufs[c % 2], bsems[c % 2])
        return (da, db)

    descs = {0: issue(0)}
    for c in range(_C_NCH):
        if c + 1 < _C_NCH:
            descs[c + 1] = issue(c + 1)
        da, db = descs[c]
        da.wait()
        db.wait()
        av = abufs[c % 2]
        bv = bbufs[c % 2]

        def add_row(i, _):
            def add_col(j, _2):
                av[i, pl.ds(j * LANES, LANES)] = (
                    av[i, pl.ds(j * LANES, LANES)]
                    + bv[i, pl.ds(j * LANES, LANES)])
                return 0
            lax.fori_loop(0, HIDDEN // LANES, add_col, 0)
            return 0

        lax.fori_loop(0, _C_CH, add_row, 0)
        pltpu.sync_copy(av, out_hbm.at[pl.ds(base + c * _C_CH, _C_CH)])


def _run_combine(out_sorted, p1, p2):
    mesh = plsc.VectorSubcoreMesh(**_SC_MESH)
    return pl.kernel(
        _combine_body,
        out_type=jax.ShapeDtypeStruct((T, HIDDEN), jnp.float32),
        mesh=mesh,
        compiler_params=pltpu.CompilerParams(needs_layout_passes=False),
        scratch_types=[
            pltpu.VMEM((_C_PER_W,), jnp.int32),
            pltpu.VMEM((_C_PER_W,), jnp.int32),
            pltpu.VMEM((_C_CH, HIDDEN), jnp.float32),
            pltpu.VMEM((_C_CH, HIDDEN), jnp.float32),
            pltpu.VMEM((_C_CH, HIDDEN), jnp.float32),
            pltpu.VMEM((_C_CH, HIDDEN), jnp.float32),
            pltpu.SemaphoreType.DMA,
            pltpu.SemaphoreType.DMA,
            pltpu.SemaphoreType.DMA,
            pltpu.SemaphoreType.DMA,
        ],
    )(out_sorted, p1, p2)


def kernel(hidden_states, gate_w, w1, w2, w3, scales):
    x = hidden_states.reshape(T, HIDDEN)
    logits, pos1, pos2, wa, wb, be, xi = _run_meta(x, gate_w)
    x_sorted, wts_sorted = _run_dispatch(
        pos1.reshape(T), pos2.reshape(T), wa.reshape(T), wb.reshape(T), xi)
    inv_s = (1.0 / scales)[:, None, :]
    out_sorted = _run_gmm(be.reshape(64), x_sorted,
                          (w1 * inv_s).astype(jnp.bfloat16),
                          (w3 * inv_s).astype(jnp.bfloat16),
                          w2.astype(jnp.bfloat16),
                          wts_sorted.reshape(NB, BLK, 1))
    final = _run_combine(out_sorted, pos1.reshape(T), pos2.reshape(T))
    return (final.reshape(B, S, HIDDEN), logits)


# speedup vs baseline: 1.2861x; 1.1435x over previous
import functools

import jax
import jax.numpy as jnp
from jax import lax
from jax.experimental import pallas as pl
from jax.experimental.pallas import tpu as pltpu
from jax.experimental.pallas import tpu_sc as plsc

B = 2
S = 2048
HIDDEN = 1024
FFN = 3584
E = 8
TOPK = 2

T = B * S
A = T * TOPK
BLK = 256
NB = A // BLK + E
NPAD = NB * BLK
NF = 1
FBLK = FFN // NF

NC = 2
NS = 16
NW = NC * NS
LANES = 16

CH = 512
NCH = A // CH

_SC_MESH = dict(core_axis_name="c", subcore_axis_name="s",
                num_cores=NC, num_subcores=NS)


def _meta_body(x_ref, gw_ref, logits_ref, pos1_ref, pos2_ref, wa_ref, wb_ref,
               be_ref, xi_ref, o_scr, r_scr):
    x = x_ref[...]
    bits = lax.bitcast_convert_type(x, jnp.int32)
    rb = lax.shift_right_logical(
        bits + 0x7FFF + jnp.bitwise_and(lax.shift_right_logical(bits, 16), 1),
        16)
    lo = rb[:, 0:HIDDEN // 2]
    hi = rb[:, HIDDEN // 2:HIDDEN]
    xi_ref[...] = jnp.bitwise_or(lo, lax.shift_left(hi, 16))
    gw = gw_ref[...]
    logits = lax.dot_general(x, gw, (((1,), (1,)), ((), ())),
                             preferred_element_type=jnp.float32)
    logits_ref[...] = logits

    p = jax.nn.softmax(logits, axis=1)
    ii = lax.broadcasted_iota(jnp.int32, (T, E), 1).astype(jnp.float32)
    m1 = jnp.max(p, axis=1, keepdims=True)
    i1 = jnp.min(jnp.where(p == m1, ii, float(E)), axis=1, keepdims=True)
    oh1 = (ii == i1).astype(jnp.float32)
    pm = jnp.where(oh1 > 0.0, -1.0, p)
    m2 = jnp.max(pm, axis=1, keepdims=True)
    i2 = jnp.min(jnp.where(pm == m2, ii, float(E)), axis=1, keepdims=True)
    oh2 = (ii == i2).astype(jnp.float32)
    ssum = m1 + m2
    wa = m1 / ssum
    wb = m2 / ssum
    wa_ref[...] = wa
    wb_ref[...] = wb

    o_scr[0:T, :] = oh1
    o_scr[T:A, :] = oh2

    tri = (lax.broadcasted_iota(jnp.int32, (CH, CH), 0)
           >= lax.broadcasted_iota(jnp.int32, (CH, CH), 1)
           ).astype(jnp.float32)

    def step(c, carry):
        blk = o_scr[pl.ds(c * CH, CH), :]
        cum = lax.dot_general(tri, blk, (((1,), (0,)), ((), ())),
                              preferred_element_type=jnp.float32) + carry
        r_scr[pl.ds(c * CH, CH), :] = jnp.sum(
            blk * (cum - 1.0), axis=1, keepdims=True)
        return carry + jnp.sum(blk, axis=0, keepdims=True)

    counts = lax.fori_loop(0, NCH, step, jnp.zeros((1, E), jnp.float32))

    bc = jnp.floor((counts + float(BLK - 1)) / float(BLK)) * float(BLK)
    lt = (lax.broadcasted_iota(jnp.int32, (E, E), 0)
          < lax.broadcasted_iota(jnp.int32, (E, E), 1)).astype(jnp.float32)
    off = lax.dot_general(bc, lt, (((1,), (0,)), ((), ())),
                          preferred_element_type=jnp.float32)

    r = r_scr[...]
    pos1 = r[0:T, :] + jnp.sum(oh1 * off, axis=1, keepdims=True)
    pos2 = r[T:A, :] + jnp.sum(oh2 * off, axis=1, keepdims=True)
    pos1_ref[...] = pos1.astype(jnp.int32)
    pos2_ref[...] = pos2.astype(jnp.int32)

    bidx = lax.broadcasted_iota(jnp.int32, (1, 64), 1).astype(jnp.float32)
    be = jnp.zeros((1, 64), jnp.float32)
    off_blocks = off / float(BLK)
    for e in range(E):
        be = be + (bidx >= off_blocks[0:1, e:e + 1]).astype(jnp.float32)
    nbu = jnp.sum(jnp.floor((counts + float(BLK - 1)) / float(BLK)),
                  axis=1, keepdims=True)
    be = jnp.where(bidx == 63.0, nbu, be - 1.0)
    be_ref[...] = be.astype(jnp.int32)


def _run_meta(x, gate_w):
    return pl.pallas_call(
        _meta_body,
        out_shape=(
            jax.ShapeDtypeStruct((T, E), jnp.float32),
            jax.ShapeDtypeStruct((T, 1), jnp.int32),
            jax.ShapeDtypeStruct((T, 1), jnp.int32),
            jax.ShapeDtypeStruct((T, 1), jnp.float32),
            jax.ShapeDtypeStruct((T, 1), jnp.float32),
            jax.ShapeDtypeStruct((1, 64), jnp.int32),
            jax.ShapeDtypeStruct((T, HIDDEN // 2), jnp.int32),
        ),
        scratch_shapes=[
            pltpu.VMEM((A, E), jnp.float32),
            pltpu.VMEM((A, 1), jnp.float32),
        ],
    )(x, gate_w)


_G_PER_W = NPAD // NW
_G_CH = 64
_G_NCH = _G_PER_W // _G_CH
HID2 = HIDDEN // 2


def _dispatch_body(pos1_hbm, pos2_hbm, wa_hbm, wb_hbm, xi_hbm,
                   xs_hbm, wts_hbm,
                   p1_v, p2_v, wa_v, wb_v, tid_v, wts_v, tid_sh,
                   idx_v, rows_v, sem):
    cid = lax.axis_index("c")
    sid = lax.axis_index("s")
    wid = sid * NC + cid

    @pl.when(sid == 0)
    def _():
        pltpu.sync_copy(pos1_hbm, p1_v)
        pltpu.sync_copy(pos2_hbm, p2_v)
        pltpu.sync_copy(wa_hbm, wa_v)
        pltpu.sync_copy(wb_hbm, wb_v)

        zi = jnp.zeros((LANES,), jnp.int32)
        zf = jnp.zeros((LANES,), jnp.float32)

        def zero(i, _):
            tid_v[pl.ds(i * LANES, LANES)] = zi
            wts_v[pl.ds(i * LANES, LANES)] = zf
            return 0

        lax.fori_loop(0, NPAD // LANES, zero, 0)

        base_iota = lax.broadcasted_iota(jnp.int32, (LANES,), 0)

        def scat(i, _):
            tok = base_iota + i * LANES
            idx1 = p1_v[pl.ds(i * LANES, LANES)]
            plsc.store_scatter(tid_v, [idx1], tok)
            plsc.store_scatter(wts_v, [idx1], wa_v[pl.ds(i * LANES, LANES)])
            idx2 = p2_v[pl.ds(i * LANES, LANES)]
            plsc.store_scatter(tid_v, [idx2], tok)
            plsc.store_scatter(wts_v, [idx2], wb_v[pl.ds(i * LANES, LANES)])
            return 0

        lax.fori_loop(0, T // LANES, scat, 0)

        pltpu.sync_copy(tid_v, tid_sh)

        @pl.when(cid == 0)
        def _():
            pltpu.sync_copy(wts_v, wts_hbm)

    plsc.subcore_barrier()

    base = wid * _G_PER_W
    pltpu.sync_copy(tid_sh.at[pl.ds(base, _G_PER_W)], idx_v)
    for c in range(_G_NCH):
        pltpu.async_copy(xi_hbm.at[idx_v.at[pl.ds(c * _G_CH, _G_CH)]],
                         rows_v, sem).wait()
        pltpu.sync_copy(rows_v, xs_hbm.at[pl.ds(base + c * _G_CH, _G_CH)])


def _run_dispatch(pos1, pos2, wa, wb, xi):
    mesh = plsc.VectorSubcoreMesh(**_SC_MESH)
    return pl.kernel(
        _dispatch_body,
        out_type=(
            jax.ShapeDtypeStruct((NPAD, HID2), jnp.int32),
            jax.ShapeDtypeStruct((NPAD,), jnp.float32),
        ),
        mesh=mesh,
        compiler_params=pltpu.CompilerParams(needs_layout_passes=False),
        scratch_types=[
            pltpu.VMEM((T,), jnp.int32),
            pltpu.VMEM((T,), jnp.int32),
            pltpu.VMEM((T,), jnp.float32),
            pltpu.VMEM((T,), jnp.float32),
            pltpu.VMEM((NPAD,), jnp.int32),
            pltpu.VMEM((NPAD,), jnp.float32),
            pltpu.VMEM_SHARED((NPAD,), jnp.int32),
            pltpu.VMEM((_G_PER_W,), jnp.int32),
            pltpu.VMEM((_G_CH, HID2), jnp.int32),
            pltpu.SemaphoreType.DMA,
        ],
    )(pos1, pos2, wa, wb, xi)


def _gmm_body(be_ref, x_ref, w1_ref, w3_ref, w2_ref, wt_ref, o_ref,
              xs_scr):
    b = pl.program_id(0)
    f = pl.program_id(1)
    nbu = be_ref[63]

    @pl.when(b < nbu)
    def _():
        @pl.when(f == 0)
        def _():
            xi = x_ref[...]
            lo = lax.bitcast_convert_type(lax.shift_left(xi, 16),
                                          jnp.float32)
            hi = lax.bitcast_convert_type(
                jnp.bitwise_and(xi, jnp.int32(-65536)), jnp.float32)
            xs_scr[...] = jnp.concatenate([lo, hi],
                                          axis=1).astype(jnp.bfloat16)

        xs = xs_scr[...]
        a = lax.dot_general(xs, w1_ref[0], (((1,), (1,)), ((), ())),
                            preferred_element_type=jnp.float32)
        g = a * jax.nn.sigmoid(a)
        c = (g * lax.dot_general(xs, w3_ref[0], (((1,), (1,)), ((), ())),
                                 preferred_element_type=jnp.float32)
             ).astype(jnp.bfloat16)
        part = lax.dot_general(c, w2_ref[0], (((1,), (1,)), ((), ())),
                               preferred_element_type=jnp.float32)

        @pl.when(f == 0)
        def _():
            o_ref[...] = part

        @pl.when(f > 0)
        def _():
            o_ref[...] = o_ref[...] + part

        @pl.when(f == NF - 1)
        def _():
            o_ref[...] = o_ref[...] * wt_ref[0]


def _run_gmm(be_vec, x_sorted, w1, w3, w2, wts3d):
    grid_spec = pltpu.PrefetchScalarGridSpec(
        num_scalar_prefetch=1,
        grid=(NB, NF),
        in_specs=[
            pl.BlockSpec((BLK, HIDDEN // 2), lambda b, f, be: (b, 0)),
            pl.BlockSpec((1, FBLK, HIDDEN), lambda b, f, be: (be[b], f, 0)),
            pl.BlockSpec((1, FBLK, HIDDEN), lambda b, f, be: (be[b], f, 0)),
            pl.BlockSpec((1, HIDDEN, FBLK), lambda b, f, be: (be[b], 0, f)),
            pl.BlockSpec((1, BLK, 1), lambda b, f, be: (b, 0, 0)),
        ],
        out_specs=pl.BlockSpec((BLK, HIDDEN), lambda b, f, be: (b, 0)),
        scratch_shapes=[pltpu.VMEM((BLK, HIDDEN), jnp.bfloat16)],
    )
    return pl.pallas_call(
        _gmm_body,
        grid_spec=grid_spec,
        out_shape=jax.ShapeDtypeStruct((NPAD, HIDDEN), jnp.float32),
        compiler_params=pltpu.CompilerParams(
            dimension_semantics=("parallel", "arbitrary"),
            vmem_limit_bytes=110 * 1024 * 1024),
    )(be_vec, x_sorted, w1, w3, w2, wts3d)


_C_PER_W = T // NW
_C_CH = 16
_C_NCH = _C_PER_W // _C_CH


def _combine_body(os_hbm, p1_hbm, p2_hbm, out_hbm, p1_v, p2_v,
                  a0_v, a1_v, b0_v, b1_v, sa0, sa1, sb0, sb1):
    cid = lax.axis_index("c")
    sid = lax.axis_index("s")
    wid = sid * NC + cid
    base = wid * _C_PER_W
    pltpu.sync_copy(p1_hbm.at[pl.ds(base, _C_PER_W)], p1_v)
    pltpu.sync_copy(p2_hbm.at[pl.ds(base, _C_PER_W)], p2_v)
    abufs = (a0_v, a1_v)
    bbufs = (b0_v, b1_v)
    asems = (sa0, sa1)
    bsems = (sb0, sb1)

    def issue(c):
        da = pltpu.async_copy(os_hbm.at[p1_v.at[pl.ds(c * _C_CH, _C_CH)]],
                              abufs[c % 2], asems[c % 2])
        db = pltpu.async_copy(os_hbm.at[p2_v.at[pl.ds(c * _C_CH, _C_CH)]],
                              bbufs[c % 2], bsems[c % 2])
        return (da, db)

    descs = {0: issue(0)}
    for c in range(_C_NCH):
        if c + 1 < _C_NCH:
            descs[c + 1] = issue(c + 1)
        da, db = descs[c]
        da.wait()
        db.wait()
        av = abufs[c % 2]
        bv = bbufs[c % 2]

        def add_row(i, _):
            def add_col(j, _2):
                av[i, pl.ds(j * LANES, LANES)] = (
                    av[i, pl.ds(j * LANES, LANES)]
                    + bv[i, pl.ds(j * LANES, LANES)])
                return 0
            lax.fori_loop(0, HIDDEN // LANES, add_col, 0)
            return 0

        lax.fori_loop(0, _C_CH, add_row, 0)
        pltpu.sync_copy(av, out_hbm.at[pl.ds(base + c * _C_CH, _C_CH)])


def _run_combine(out_sorted, p1, p2):
    mesh = plsc.VectorSubcoreMesh(**_SC_MESH)
    return pl.kernel(
        _combine_body,
        out_type=jax.ShapeDtypeStruct((T, HIDDEN), jnp.float32),
        mesh=mesh,
        compiler_params=pltpu.CompilerParams(needs_layout_passes=False),
        scratch_types=[
            pltpu.VMEM((_C_PER_W,), jnp.int32),
            pltpu.VMEM((_C_PER_W,), jnp.int32),
            pltpu.VMEM((_C_CH, HIDDEN), jnp.float32),
            pltpu.VMEM((_C_CH, HIDDEN), jnp.float32),
            pltpu.VMEM((_C_CH, HIDDEN), jnp.float32),
            pltpu.VMEM((_C_CH, HIDDEN), jnp.float32),
            pltpu.SemaphoreType.DMA,
            pltpu.SemaphoreType.DMA,
            pltpu.SemaphoreType.DMA,
            pltpu.SemaphoreType.DMA,
        ],
    )(out_sorted, p1, p2)


def kernel(hidden_states, gate_w, w1, w2, w3, scales):
    x = hidden_states.reshape(T, HIDDEN)
    logits, pos1, pos2, wa, wb, be, xi = _run_meta(x, gate_w)
    x_sorted, wts_sorted = _run_dispatch(
        pos1.reshape(T), pos2.reshape(T), wa.reshape(T), wb.reshape(T), xi)
    inv_s = (1.0 / scales)[:, None, :]
    out_sorted = _run_gmm(be.reshape(64), x_sorted,
                          (w1 * inv_s).astype(jnp.bfloat16),
                          (w3 * inv_s).astype(jnp.bfloat16),
                          w2.astype(jnp.bfloat16),
                          wts_sorted.reshape(NB, BLK, 1))
    final = _run_combine(out_sorted, pos1.reshape(T), pos2.reshape(T))
    return (final.reshape(B, S, HIDDEN), logits)
